# baseline (speedup 1.0000x reference)
import jax
import jax.numpy as jnp
from jax.experimental import pallas as pl
from jax.experimental.pallas import tpu as pltpu

GAMMA = 0.1
BLOCK_B = 512
CH = 4
NSLOT = 4
LA = NSLOT - 1


def _body(s_ref, w_ref, b_ref, unif_hbm, mask_hbm, out_hbm,
          ubuf, mbuf, obuf, usem, msem, osem):
    i = pl.program_id(0)
    n = pl.num_programs(0)
    slot = jax.lax.rem(i, NSLOT)
    rows = BLOCK_B // CH

    def in_copy(block, slot, c, start):
        base = block * BLOCK_B + c * rows
        u = pltpu.make_async_copy(
            unif_hbm.at[pl.ds(base, rows), :],
            ubuf.at[slot, pl.ds(c * rows, rows), :],
            usem.at[slot, c])
        m = pltpu.make_async_copy(
            mask_hbm.at[pl.ds(base, rows), :],
            mbuf.at[slot, pl.ds(c * rows, rows), :],
            msem.at[slot, c])
        if start:
            u.start()
            m.start()
        else:
            u.wait()
            m.wait()

    def out_copy(block, slot, c, start):
        base = block * BLOCK_B + c * rows
        o = pltpu.make_async_copy(
            obuf.at[slot, pl.ds(c * rows, rows), :],
            out_hbm.at[pl.ds(base, rows), :],
            osem.at[slot, c])
        if start:
            o.start()
        else:
            o.wait()

    @pl.when(i == 0)
    def _():
        for blk in range(LA + 1):
            for c in range(CH):
                in_copy(blk, blk % NSLOT, c, start=True)

    @pl.when(jnp.logical_and(i > 0, i + LA < n))
    def _():
        for c in range(CH):
            in_copy(i + LA, jax.lax.rem(i + LA, NSLOT), c, start=True)

    for c in range(CH):
        in_copy(i, slot, c, start=False)

    @pl.when(i >= NSLOT)
    def _():
        for c in range(CH):
            out_copy(i - NSLOT, slot, c, start=False)

    logits = jnp.dot(s_ref[...], w_ref[...], preferred_element_type=jnp.float32)
    logits = logits + b_ref[...]
    mx = jnp.max(logits, axis=1, keepdims=True)
    e = jnp.exp(logits - mx)
    denom = jnp.sum(e, axis=1, keepdims=True)
    probs = GAMMA * ubuf[slot] + ((1.0 - GAMMA) / denom) * e
    a = logits.shape[1]
    col = jax.lax.broadcasted_iota(jnp.int32, logits.shape, 1)
    valid = jnp.logical_or(mbuf[slot] != 0, col == a - 1)
    probs = jnp.where(valid, probs, 0.0)
    obuf[slot] = probs * (1.0 / jnp.sum(probs, axis=1, keepdims=True))

    for c in range(CH):
        out_copy(i, slot, c, start=True)

    @pl.when(i == n - 1)
    def _():
        for k in range(1, NSLOT + 1):
            blk = i - NSLOT + k
            for c in range(CH):
                out_copy(blk, jax.lax.rem(blk + NSLOT, NSLOT), c, start=False)


@jax.jit
def kernel(s, unif, mask, W, b):
    bsz, d = s.shape
    a = W.shape[1]
    n = bsz // BLOCK_B
    return pl.pallas_call(
        _body,
        grid=(n,),
        in_specs=[
            pl.BlockSpec((BLOCK_B, d), lambda i: (i, 0)),
            pl.BlockSpec((d, a), lambda i: (0, 0)),
            pl.BlockSpec((1, a), lambda i: (0, 0)),
            pl.BlockSpec(memory_space=pl.ANY),
            pl.BlockSpec(memory_space=pl.ANY),
        ],
        out_specs=pl.BlockSpec(memory_space=pl.ANY),
        out_shape=jax.ShapeDtypeStruct((bsz, a), jnp.float32),
        scratch_shapes=[
            pltpu.VMEM((NSLOT, BLOCK_B, a), jnp.float32),
            pltpu.VMEM((NSLOT, BLOCK_B, a), jnp.int32),
            pltpu.VMEM((NSLOT, BLOCK_B, a), jnp.float32),
            pltpu.SemaphoreType.DMA((NSLOT, CH)),
            pltpu.SemaphoreType.DMA((NSLOT, CH)),
            pltpu.SemaphoreType.DMA((NSLOT, CH)),
        ],
        compiler_params=pltpu.CompilerParams(
            dimension_semantics=("arbitrary",),
        ),
    )(s, W, b.reshape(1, a), unif, mask)
